# Initial kernel scaffold; baseline (speedup 1.0000x reference)
import jax
import jax.numpy as jnp
from jax.experimental import pallas as pl


def kernel(inputs):
    raise NotImplementedError("write your pallas kernel here")


# speedup vs baseline: 24.5464x; 24.5464x over previous
import functools

import jax
import jax.numpy as jnp
from jax import lax
from jax.experimental import pallas as pl
from jax.experimental.pallas import tpu as pltpu
from jax.experimental.pallas import tpu_sc as plsc

_B, _H, _C = 32, 32768, 16
_K = 8
_NC, _NS = 2, 16
_W = 8 * _C
_R = _H // 8
_CHUNK = 256
_NCHUNK = _R // _CHUNK

_SORT8 = (
    (0, 1), (2, 3), (4, 5), (6, 7),
    (0, 2), (1, 3), (4, 6), (5, 7),
    (1, 2), (5, 6),
    (0, 4), (1, 5), (2, 6), (3, 7),
    (2, 4), (3, 5),
    (1, 2), (3, 4), (5, 6),
)
_MERGE8 = (
    (0, 4), (1, 5), (2, 6), (3, 7),
    (0, 2), (1, 3), (4, 6), (5, 7),
    (0, 1), (2, 3), (4, 5), (6, 7),
)


def _sort8(v):
    v = list(v)
    for i, j in _SORT8:
        hi = jnp.maximum(v[i], v[j])
        lo = jnp.minimum(v[i], v[j])
        v[i], v[j] = hi, lo
    return v


def _merge_top8(r, g):
    w = [jnp.maximum(r[i], g[7 - i]) for i in range(8)]
    for i, j in _MERGE8:
        hi = jnp.maximum(w[i], w[j])
        lo = jnp.minimum(w[i], w[j])
        w[i], w[j] = hi, lo
    return w


@functools.partial(
    pl.kernel,
    out_type=jax.ShapeDtypeStruct((_B, _K * _C), jnp.float32),
    mesh=plsc.VectorSubcoreMesh(core_axis_name="c", subcore_axis_name="s"),
    scratch_types=[
        pltpu.VMEM((_CHUNK, _W), jnp.float32),
        pltpu.VMEM((_CHUNK, _W), jnp.float32),
        pltpu.VMEM((1, _W), jnp.float32),
        pltpu.SemaphoreType.DMA,
        pltpu.SemaphoreType.DMA,
    ],
)
def _kmax_sc(x_hbm, out_hbm, buf0, buf1, out_v, sem0, sem1):
    wid = lax.axis_index("s") * _NC + lax.axis_index("c")
    bufs = (buf0, buf1)
    sems = (sem0, sem1)

    copies = {}
    copies[0] = pltpu.async_copy(
        x_hbm.at[wid, pl.ds(0, _CHUNK)], buf0, sem0)

    r = [jnp.full((_C,), -jnp.inf, jnp.float32) for _ in range(_K)]
    for ci in range(_NCHUNK):
        buf = bufs[ci % 2]
        copies.pop(ci).wait()
        if ci + 1 < _NCHUNK:
            copies[ci + 1] = pltpu.async_copy(
                x_hbm.at[wid, pl.ds((ci + 1) * _CHUNK, _CHUNK)],
                bufs[(ci + 1) % 2], sems[(ci + 1) % 2])

        def body(g, rs, buf=buf):
            v = [buf[g, pl.ds(k * _C, _C)] for k in range(8)]
            v = _sort8(v)
            return tuple(_merge_top8(list(rs), v))

        r = list(lax.fori_loop(0, _CHUNK, body, tuple(r)))

    for i in range(_K):
        out_v[0, pl.ds(i * _C, _C)] = r[i]
    pltpu.sync_copy(out_v, out_hbm.at[pl.ds(wid, 1)])


def kernel(inputs):
    x = inputs.reshape(_B, _R, _W)
    out = _kmax_sc(x)
    return out.reshape(_B, _K, 1, _C)
